# Initial kernel scaffold; baseline (speedup 1.0000x reference)
import jax
import jax.numpy as jnp
from jax.experimental import pallas as pl


def kernel(index, rate, starttime, endtime, t):
    raise NotImplementedError("write your pallas kernel here")


# speedup vs baseline: 25.6156x; 25.6156x over previous
import functools

import jax
import jax.numpy as jnp
from jax import lax
from jax.experimental import pallas as pl
from jax.experimental.pallas import tpu as pltpu
from jax.experimental.pallas import tpu_sc as plsc

SIZE = 100000
HALF = 50000
NEV = 3200000
POOLW = 10
NBUCK = 5000
PSTR = 5120
ACC_N = 65536
ROWW = 128
ACC_ROWS = ACC_N // ROWW
NW = 32
EW = NEV // NW
CHUNK = 2000
NCH = EW // CHUNK
GROUPS = CHUNK // 16


def _sc_partials(index, rate, starttime, endtime, tvec):
    mesh = plsc.VectorSubcoreMesh(core_axis_name="c", subcore_axis_name="s")

    @functools.partial(
        pl.kernel,
        mesh=mesh,
        out_type=jax.ShapeDtypeStruct((NW, ACC_N), jnp.float32),
        scratch_types=[
            pltpu.VMEM((CHUNK,), jnp.int32),
            pltpu.VMEM((CHUNK,), jnp.int32),
            pltpu.VMEM((CHUNK,), jnp.float32),
            pltpu.VMEM((CHUNK,), jnp.float32),
            pltpu.VMEM((CHUNK,), jnp.float32),
            pltpu.VMEM((CHUNK,), jnp.float32),
            pltpu.VMEM((CHUNK,), jnp.float32),
            pltpu.VMEM((CHUNK,), jnp.float32),
            pltpu.VMEM((ACC_N,), jnp.float32),
            pltpu.VMEM((16,), jnp.float32),
        ] + [pltpu.SemaphoreType.DMA] * 8,
        compiler_params=pltpu.CompilerParams(needs_layout_passes=False),
    )
    def body(idx_h, rate_h, st_h, en_h, tv_h, out_h,
             ib0, ib1, rb0, rb1, sb0, sb1, eb0, eb1, acc, tb, *sems):
        ib = (ib0, ib1)
        rb = (rb0, rb1)
        sb = (sb0, sb1)
        eb = (eb0, eb1)
        cid = lax.axis_index("c")
        sid = lax.axis_index("s")
        wid = cid * 16 + sid
        base = wid * EW

        def copies(slot, g):
            off = base + g * CHUNK
            return [
                pltpu.make_async_copy(idx_h.at[pl.ds(off, CHUNK)],
                                      ib[slot], sems[slot * 4 + 0]),
                pltpu.make_async_copy(rate_h.at[pl.ds(off, CHUNK)],
                                      rb[slot], sems[slot * 4 + 1]),
                pltpu.make_async_copy(st_h.at[pl.ds(off, CHUNK)],
                                      sb[slot], sems[slot * 4 + 2]),
                pltpu.make_async_copy(en_h.at[pl.ds(off, CHUNK)],
                                      eb[slot], sems[slot * 4 + 3]),
            ]

        def start(slot, g):
            for c in copies(slot, g):
                c.start()

        def wait(slot, g):
            for c in copies(slot, g):
                c.wait()

        zero16 = jnp.zeros((16,), jnp.float32)

        def zacc(i, carry):
            for u in range(8):
                acc[pl.ds((i * 8 + u) * 16, 16)] = zero16
            return carry

        lax.fori_loop(0, ACC_N // 128, zacc, 0)

        pltpu.sync_copy(tv_h, tb)
        tv = tb[...]

        start(0, 0)

        def outer(it, carry):
            g0 = it * 2
            for b in range(2):
                g = g0 + b

                wait(b, g)

                @pl.when(g + 1 < NCH)
                def _():
                    start(1 - b, g + 1)

                def grp(i, c2):
                    idx = ib[b][pl.ds(i * 16, 16)]
                    rt = rb[b][pl.ds(i * 16, 16)]
                    st = sb[b][pl.ds(i * 16, 16)]
                    en = eb[b][pl.ds(i * 16, 16)]
                    m = (st <= tv) & (tv < en)
                    val = jnp.where(m, rt, 0.0)
                    hm = idx < HALF
                    n = jnp.where(hm, idx, idx - HALF)
                    q = ((n.astype(jnp.uint32) * jnp.uint32(52429))
                         >> jnp.uint32(19)).astype(jnp.int32)
                    w = n - q * 10
                    pos = jnp.where(hm, q, PSTR + w * PSTR + q)
                    plsc.addupdate_scatter(acc, [pos], val)
                    return c2

                lax.fori_loop(0, GROUPS, grp, 0)
            return carry

        lax.fori_loop(0, NCH // 2, outer, 0)

        pltpu.sync_copy(acc, out_h.at[wid])

    return body(index, rate, starttime, endtime, tvec)


def _tc_combine(partials):
    def body(p_ref, sum_ref, max_ref):
        a = p_ref[0]
        for k in range(1, NW):
            a = a + p_ref[k]
        sum_ref[...] = a[0:40, :]
        m = a[40:80, :]
        for w in range(1, POOLW):
            m = jnp.maximum(m, a[40 * (w + 1):40 * (w + 2), :])
        max_ref[...] = m

    return pl.pallas_call(
        body,
        out_shape=[jax.ShapeDtypeStruct((40, ROWW), jnp.float32),
                   jax.ShapeDtypeStruct((40, ROWW), jnp.float32)],
    )(partials)


@jax.jit
def kernel(index, rate, starttime, endtime, t):
    tvec = jnp.full((16,), t, jnp.float32)
    parts = _sc_partials(index.astype(jnp.int32), rate, starttime,
                         endtime, tvec)
    s, m = _tc_combine(parts.reshape(NW, ACC_ROWS, ROWW))
    return jnp.concatenate([s.reshape(-1)[:NBUCK], m.reshape(-1)[:NBUCK]])


# speedup vs baseline: 38.1312x; 1.4886x over previous
import functools

import jax
import jax.numpy as jnp
from jax import lax
from jax.experimental import pallas as pl
from jax.experimental.pallas import tpu as pltpu
from jax.experimental.pallas import tpu_sc as plsc

SIZE = 100000
HALF = 50000
NEV = 3200000
POOLW = 10
NBUCK = 5000
PSTR = 5120
ACC_N = 65536
ROWW = 128
ACC_ROWS = ACC_N // ROWW
NW = 32
EW = NEV // NW
CHUNK = 2000
NCH = EW // CHUNK
GROUPS = CHUNK // 16


def _sc_partials(index, rate, starttime, endtime, tvec):
    mesh = plsc.VectorSubcoreMesh(core_axis_name="c", subcore_axis_name="s")

    @functools.partial(
        pl.kernel,
        mesh=mesh,
        out_type=jax.ShapeDtypeStruct((NW, ACC_N), jnp.float32),
        scratch_types=[
            pltpu.VMEM((CHUNK,), jnp.int32),
            pltpu.VMEM((CHUNK,), jnp.int32),
            pltpu.VMEM((CHUNK,), jnp.float32),
            pltpu.VMEM((CHUNK,), jnp.float32),
            pltpu.VMEM((CHUNK,), jnp.float32),
            pltpu.VMEM((CHUNK,), jnp.float32),
            pltpu.VMEM((CHUNK,), jnp.float32),
            pltpu.VMEM((CHUNK,), jnp.float32),
            pltpu.VMEM((ACC_N,), jnp.float32),
            pltpu.VMEM((16,), jnp.float32),
        ] + [pltpu.SemaphoreType.DMA] * 8,
        compiler_params=pltpu.CompilerParams(needs_layout_passes=False),
    )
    def body(idx_h, rate_h, st_h, en_h, tv_h, out_h,
             ib0, ib1, rb0, rb1, sb0, sb1, eb0, eb1, acc, tb, *sems):
        ib = (ib0, ib1)
        rb = (rb0, rb1)
        sb = (sb0, sb1)
        eb = (eb0, eb1)
        cid = lax.axis_index("c")
        sid = lax.axis_index("s")
        wid = cid * 16 + sid
        base = wid * EW

        def copies(slot, g):
            off = base + g * CHUNK
            return [
                pltpu.make_async_copy(idx_h.at[pl.ds(off, CHUNK)],
                                      ib[slot], sems[slot * 4 + 0]),
                pltpu.make_async_copy(rate_h.at[pl.ds(off, CHUNK)],
                                      rb[slot], sems[slot * 4 + 1]),
                pltpu.make_async_copy(st_h.at[pl.ds(off, CHUNK)],
                                      sb[slot], sems[slot * 4 + 2]),
                pltpu.make_async_copy(en_h.at[pl.ds(off, CHUNK)],
                                      eb[slot], sems[slot * 4 + 3]),
            ]

        def start(slot, g):
            for c in copies(slot, g):
                c.start()

        def wait(slot, g):
            for c in copies(slot, g):
                c.wait()

        zero16 = jnp.zeros((16,), jnp.float32)

        @plsc.parallel_loop(0, ACC_N // 16, unroll=8)
        def zacc(i):
            acc[pl.ds(i * 16, 16)] = zero16

        pltpu.sync_copy(tv_h, tb)
        tv = tb[...]

        start(0, 0)

        def outer(it, carry):
            g0 = it * 2
            for b in range(2):
                g = g0 + b

                wait(b, g)

                @pl.when(g + 1 < NCH)
                def _():
                    start(1 - b, g + 1)

                @plsc.parallel_loop(0, GROUPS, unroll=4)
                def grp(i):
                    idx = ib[b][pl.ds(i * 16, 16)]
                    rt = rb[b][pl.ds(i * 16, 16)]
                    st = sb[b][pl.ds(i * 16, 16)]
                    en = eb[b][pl.ds(i * 16, 16)]
                    m = (st <= tv) & (tv < en)
                    val = jnp.where(m, rt, 0.0)
                    hm = idx < HALF
                    n = jnp.where(hm, idx, idx - HALF)
                    q = ((n.astype(jnp.uint32) * jnp.uint32(52429))
                         >> jnp.uint32(19)).astype(jnp.int32)
                    w = n - q * 10
                    pos = jnp.where(hm, q, PSTR + w * PSTR + q)
                    plsc.addupdate_scatter(acc, [pos], val)
            return carry

        lax.fori_loop(0, NCH // 2, outer, 0)

        pltpu.sync_copy(acc, out_h.at[wid])

    return body(index, rate, starttime, endtime, tvec)


def _tc_combine(partials):
    def body(p_ref, sum_ref, max_ref):
        a = p_ref[0]
        for k in range(1, NW):
            a = a + p_ref[k]
        sum_ref[...] = a[0:40, :]
        m = a[40:80, :]
        for w in range(1, POOLW):
            m = jnp.maximum(m, a[40 * (w + 1):40 * (w + 2), :])
        max_ref[...] = m

    return pl.pallas_call(
        body,
        out_shape=[jax.ShapeDtypeStruct((40, ROWW), jnp.float32),
                   jax.ShapeDtypeStruct((40, ROWW), jnp.float32)],
    )(partials)


@jax.jit
def kernel(index, rate, starttime, endtime, t):
    tvec = jnp.full((16,), t, jnp.float32)
    parts = _sc_partials(index.astype(jnp.int32), rate, starttime,
                         endtime, tvec)
    s, m = _tc_combine(parts.reshape(NW, ACC_ROWS, ROWW))
    return jnp.concatenate([s.reshape(-1)[:NBUCK], m.reshape(-1)[:NBUCK]])


# speedup vs baseline: 52.4528x; 1.3756x over previous
import functools

import jax
import jax.numpy as jnp
from jax import lax
from jax.experimental import pallas as pl
from jax.experimental.pallas import tpu as pltpu
from jax.experimental.pallas import tpu_sc as plsc

SIZE = 100000
HALF = 50000
NEV = 3200000
POOLW = 10
NBUCK = 5000
PSTR = 5120
ACC_N = 65536
ROWW = 128
ACC_ROWS = ACC_N // ROWW
NW = 32
EW = NEV // NW
CHUNK = 4000
NCH = EW // CHUNK
GROUPS = CHUNK // 16


def _sc_partials(index, rate, starttime, endtime, tvec):
    mesh = plsc.VectorSubcoreMesh(core_axis_name="c", subcore_axis_name="s")

    @functools.partial(
        pl.kernel,
        mesh=mesh,
        out_type=jax.ShapeDtypeStruct((NW, ACC_N), jnp.float32),
        scratch_types=[
            pltpu.VMEM((CHUNK,), jnp.int32),
            pltpu.VMEM((CHUNK,), jnp.int32),
            pltpu.VMEM((CHUNK,), jnp.float32),
            pltpu.VMEM((CHUNK,), jnp.float32),
            pltpu.VMEM((CHUNK,), jnp.float32),
            pltpu.VMEM((CHUNK,), jnp.float32),
            pltpu.VMEM((CHUNK,), jnp.float32),
            pltpu.VMEM((CHUNK,), jnp.float32),
            pltpu.VMEM((ACC_N,), jnp.float32),
            pltpu.VMEM((16,), jnp.float32),
        ] + [pltpu.SemaphoreType.DMA] * 8,
        compiler_params=pltpu.CompilerParams(needs_layout_passes=False),
    )
    def body(idx_h, rate_h, st_h, en_h, tv_h, out_h,
             ib0, ib1, rb0, rb1, sb0, sb1, eb0, eb1, acc, tb, *sems):
        ib = (ib0, ib1)
        rb = (rb0, rb1)
        sb = (sb0, sb1)
        eb = (eb0, eb1)
        cid = lax.axis_index("c")
        sid = lax.axis_index("s")
        wid = cid * 16 + sid
        base = wid * EW

        def copies(slot, g):
            off = base + g * CHUNK
            return [
                pltpu.make_async_copy(idx_h.at[pl.ds(off, CHUNK)],
                                      ib[slot], sems[slot * 4 + 0]),
                pltpu.make_async_copy(rate_h.at[pl.ds(off, CHUNK)],
                                      rb[slot], sems[slot * 4 + 1]),
                pltpu.make_async_copy(st_h.at[pl.ds(off, CHUNK)],
                                      sb[slot], sems[slot * 4 + 2]),
                pltpu.make_async_copy(en_h.at[pl.ds(off, CHUNK)],
                                      eb[slot], sems[slot * 4 + 3]),
            ]

        def start(slot, g):
            for c in copies(slot, g):
                c.start()

        def wait(slot, g):
            for c in copies(slot, g):
                c.wait()

        zero16 = jnp.zeros((16,), jnp.float32)

        @plsc.parallel_loop(0, ACC_N // 16, unroll=8)
        def zacc(i):
            acc[pl.ds(i * 16, 16)] = zero16

        pltpu.sync_copy(tv_h, tb)
        tv = tb[...]

        start(0, 0)

        def chunk_compute(b):
            @plsc.parallel_loop(0, GROUPS, unroll=4)
            def grp(i):
                idx = ib[b][pl.ds(i * 16, 16)]
                rt = rb[b][pl.ds(i * 16, 16)]
                st = sb[b][pl.ds(i * 16, 16)]
                en = eb[b][pl.ds(i * 16, 16)]
                m = (st <= tv) & (tv < en)
                val = jnp.where(m, rt, 0.0)
                hm = idx < HALF
                n = jnp.where(hm, idx, idx - HALF)
                q = ((n.astype(jnp.uint32) * jnp.uint32(52429))
                     >> jnp.uint32(19)).astype(jnp.int32)
                w = n - q * 10
                pos = jnp.where(hm, q, PSTR + w * PSTR + q)
                plsc.addupdate_scatter(acc, [pos], val)

        def outer(it, carry):
            g0 = it * 2
            for b in range(2):
                g = g0 + b
                wait(b, g)

                @pl.when(g + 1 < NCH)
                def _():
                    start(1 - b, g + 1)

                chunk_compute(b)
            return carry

        lax.fori_loop(0, (NCH - 1) // 2, outer, 0)
        wait(0, NCH - 1)
        chunk_compute(0)

        pltpu.sync_copy(acc, out_h.at[wid])

    return body(index, rate, starttime, endtime, tvec)


def _tc_combine(partials):
    def body(p_ref, sum_ref, max_ref):
        a = p_ref[0]
        for k in range(1, NW):
            a = a + p_ref[k]
        sum_ref[...] = a[0:PSTR]
        m = a[PSTR:2 * PSTR]
        for w in range(1, POOLW):
            m = jnp.maximum(m, a[PSTR * (w + 1):PSTR * (w + 2)])
        max_ref[...] = m

    return pl.pallas_call(
        body,
        out_shape=[jax.ShapeDtypeStruct((PSTR,), jnp.float32),
                   jax.ShapeDtypeStruct((PSTR,), jnp.float32)],
    )(partials)


@jax.jit
def kernel(index, rate, starttime, endtime, t):
    tvec = jnp.full((16,), t, jnp.float32)
    parts = _sc_partials(index.astype(jnp.int32), rate, starttime,
                         endtime, tvec)
    s, m = _tc_combine(parts)
    return jnp.concatenate([s[:NBUCK], m[:NBUCK]])


# speedup vs baseline: 53.3224x; 1.0166x over previous
import functools

import jax
import jax.numpy as jnp
from jax import lax
from jax.experimental import pallas as pl
from jax.experimental.pallas import tpu as pltpu
from jax.experimental.pallas import tpu_sc as plsc

SIZE = 100000
HALF = 50000
NEV = 3200000
POOLW = 10
NBUCK = 5000
PSTR = 5120
ACC_N = 65536
ROWW = 128
ACC_ROWS = ACC_N // ROWW
NW = 32
EW = NEV // NW
CHUNK = 4000
NCH = EW // CHUNK
GROUPS = CHUNK // 16


def _sc_partials(index, rate, starttime, endtime, tvec):
    mesh = plsc.VectorSubcoreMesh(core_axis_name="c", subcore_axis_name="s")

    @functools.partial(
        pl.kernel,
        mesh=mesh,
        out_type=jax.ShapeDtypeStruct((NW, ACC_N), jnp.float32),
        scratch_types=[
            pltpu.VMEM((CHUNK,), jnp.int32),
            pltpu.VMEM((CHUNK,), jnp.int32),
            pltpu.VMEM((CHUNK,), jnp.float32),
            pltpu.VMEM((CHUNK,), jnp.float32),
            pltpu.VMEM((CHUNK,), jnp.float32),
            pltpu.VMEM((CHUNK,), jnp.float32),
            pltpu.VMEM((CHUNK,), jnp.float32),
            pltpu.VMEM((CHUNK,), jnp.float32),
            pltpu.VMEM((ACC_N,), jnp.float32),
            pltpu.VMEM((16,), jnp.float32),
        ] + [pltpu.SemaphoreType.DMA] * 8,
        compiler_params=pltpu.CompilerParams(needs_layout_passes=False),
    )
    def body(idx_h, rate_h, st_h, en_h, tv_h, out_h,
             ib0, ib1, rb0, rb1, sb0, sb1, eb0, eb1, acc, tb, *sems):
        ib = (ib0, ib1)
        rb = (rb0, rb1)
        sb = (sb0, sb1)
        eb = (eb0, eb1)
        cid = lax.axis_index("c")
        sid = lax.axis_index("s")
        wid = cid * 16 + sid
        base = wid * EW

        def copies(slot, g):
            off = base + g * CHUNK
            return [
                pltpu.make_async_copy(idx_h.at[pl.ds(off, CHUNK)],
                                      ib[slot], sems[slot * 4 + 0]),
                pltpu.make_async_copy(rate_h.at[pl.ds(off, CHUNK)],
                                      rb[slot], sems[slot * 4 + 1]),
                pltpu.make_async_copy(st_h.at[pl.ds(off, CHUNK)],
                                      sb[slot], sems[slot * 4 + 2]),
                pltpu.make_async_copy(en_h.at[pl.ds(off, CHUNK)],
                                      eb[slot], sems[slot * 4 + 3]),
            ]

        def start(slot, g):
            for c in copies(slot, g):
                c.start()

        def wait(slot, g):
            for c in copies(slot, g):
                c.wait()

        start(0, 0)
        pltpu.sync_copy(tv_h, tb)
        tv = tb[...]

        zero16 = jnp.zeros((16,), jnp.float32)

        @plsc.parallel_loop(0, ACC_N // 16, unroll=8)
        def zacc(i):
            acc[pl.ds(i * 16, 16)] = zero16

        def chunk_compute(b):
            @plsc.parallel_loop(0, GROUPS, unroll=4)
            def grp(i):
                idx = ib[b][pl.ds(i * 16, 16)]
                rt = rb[b][pl.ds(i * 16, 16)]
                st = sb[b][pl.ds(i * 16, 16)]
                en = eb[b][pl.ds(i * 16, 16)]
                m = (st <= tv) & (tv < en)
                hm = idx < HALF
                n = jnp.where(hm, idx, idx - HALF)
                q = ((n.astype(jnp.uint32) * jnp.uint32(52429))
                     >> jnp.uint32(19)).astype(jnp.int32)
                off2 = n * PSTR + PSTR - q * (10 * PSTR)
                pos = q + jnp.where(hm, 0, off2)
                plsc.addupdate_scatter(acc, [pos], rt, mask=m)

        def outer(it, carry):
            g0 = it * 2
            for b in range(2):
                g = g0 + b
                wait(b, g)

                @pl.when(g + 1 < NCH)
                def _():
                    start(1 - b, g + 1)

                chunk_compute(b)
            return carry

        lax.fori_loop(0, (NCH - 1) // 2, outer, 0)
        wait(0, NCH - 1)
        chunk_compute(0)

        pltpu.sync_copy(acc, out_h.at[wid])

    return body(index, rate, starttime, endtime, tvec)


def _tc_combine(partials):
    def body(p_ref, sum_ref, max_ref):
        a = p_ref[0]
        for k in range(1, NW):
            a = a + p_ref[k]
        sum_ref[...] = a[0:PSTR]
        m = a[PSTR:2 * PSTR]
        for w in range(1, POOLW):
            m = jnp.maximum(m, a[PSTR * (w + 1):PSTR * (w + 2)])
        max_ref[...] = m

    return pl.pallas_call(
        body,
        out_shape=[jax.ShapeDtypeStruct((PSTR,), jnp.float32),
                   jax.ShapeDtypeStruct((PSTR,), jnp.float32)],
    )(partials)


@jax.jit
def kernel(index, rate, starttime, endtime, t):
    tvec = jnp.full((16,), t, jnp.float32)
    parts = _sc_partials(index.astype(jnp.int32), rate, starttime,
                         endtime, tvec)
    s, m = _tc_combine(parts)
    return jnp.concatenate([s[:NBUCK], m[:NBUCK]])


# speedup vs baseline: 62.2291x; 1.1670x over previous
import functools

import jax
import jax.numpy as jnp
from jax import lax
from jax.experimental import pallas as pl
from jax.experimental.pallas import tpu as pltpu
from jax.experimental.pallas import tpu_sc as plsc

SIZE = 100000
HALF = 50000
NEV = 3200000
POOLW = 10
NBUCK = 5000
PSTR = 5120
ACC_N = 65536
ROWW = 128
ACC_ROWS = ACC_N // ROWW
NW = 32
EW = NEV // NW
CHUNK = 4000
NCH = EW // CHUNK
GROUPS = CHUNK // 16
NSLOT = 3


def _sc_partials(index, rate, starttime, endtime, tvec):
    mesh = plsc.VectorSubcoreMesh(core_axis_name="c", subcore_axis_name="s")

    @functools.partial(
        pl.kernel,
        mesh=mesh,
        out_type=jax.ShapeDtypeStruct((NW, ACC_N), jnp.float32),
        scratch_types=(
            [pltpu.VMEM((CHUNK,), jnp.int32)] * NSLOT
            + [pltpu.VMEM((CHUNK,), jnp.float32)] * (3 * NSLOT)
            + [
                pltpu.VMEM((ACC_N,), jnp.float32),
                pltpu.VMEM((16,), jnp.float32),
            ]
            + [pltpu.SemaphoreType.DMA] * (4 * NSLOT)
        ),
        compiler_params=pltpu.CompilerParams(needs_layout_passes=False),
    )
    def body(idx_h, rate_h, st_h, en_h, tv_h, out_h, *scratch):
        ib = scratch[0:NSLOT]
        rb = scratch[NSLOT:2 * NSLOT]
        sb = scratch[2 * NSLOT:3 * NSLOT]
        eb = scratch[3 * NSLOT:4 * NSLOT]
        acc, tb = scratch[4 * NSLOT], scratch[4 * NSLOT + 1]
        sems = scratch[4 * NSLOT + 2:]
        cid = lax.axis_index("c")
        sid = lax.axis_index("s")
        wid = cid * 16 + sid
        base = wid * EW

        def copies(slot, g):
            off = base + g * CHUNK
            return [
                pltpu.make_async_copy(idx_h.at[pl.ds(off, CHUNK)],
                                      ib[slot], sems[slot * 4 + 0]),
                pltpu.make_async_copy(rate_h.at[pl.ds(off, CHUNK)],
                                      rb[slot], sems[slot * 4 + 1]),
                pltpu.make_async_copy(st_h.at[pl.ds(off, CHUNK)],
                                      sb[slot], sems[slot * 4 + 2]),
                pltpu.make_async_copy(en_h.at[pl.ds(off, CHUNK)],
                                      eb[slot], sems[slot * 4 + 3]),
            ]

        def start(slot, g):
            for c in copies(slot, g):
                c.start()

        def wait(slot, g):
            for c in copies(slot, g):
                c.wait()

        start(0, 0)
        start(1, 1)
        pltpu.sync_copy(tv_h, tb)
        tv = tb[...]

        zero16 = jnp.zeros((16,), jnp.float32)

        @plsc.parallel_loop(0, ACC_N // 16, unroll=8)
        def zacc(i):
            acc[pl.ds(i * 16, 16)] = zero16

        def chunk_compute(b):
            @plsc.parallel_loop(0, GROUPS, unroll=4)
            def grp(i):
                idx = ib[b][pl.ds(i * 16, 16)]
                rt = rb[b][pl.ds(i * 16, 16)]
                st = sb[b][pl.ds(i * 16, 16)]
                en = eb[b][pl.ds(i * 16, 16)]
                m = (st <= tv) & (tv < en)
                hm = idx < HALF
                n = jnp.where(hm, idx, idx - HALF)
                q = ((n.astype(jnp.uint32) * jnp.uint32(52429))
                     >> jnp.uint32(19)).astype(jnp.int32)
                off2 = n * PSTR + PSTR - q * (10 * PSTR)
                pos = q + jnp.where(hm, 0, off2)
                plsc.addupdate_scatter(acc, [pos], rt, mask=m)

        def outer(it, carry):
            g0 = it * NSLOT
            for b in range(NSLOT):
                g = g0 + b
                wait(b, g)

                @pl.when(g + 2 < NCH)
                def _():
                    start((b + 2) % NSLOT, g + 2)

                chunk_compute(b)
            return carry

        lax.fori_loop(0, (NCH - 1) // NSLOT, outer, 0)
        wait((NCH - 1) % NSLOT, NCH - 1)
        chunk_compute((NCH - 1) % NSLOT)

        pltpu.sync_copy(acc, out_h.at[wid])

    return body(index, rate, starttime, endtime, tvec)


def _tc_combine(partials):
    def body(p_ref, sum_ref, max_ref):
        a = p_ref[0]
        for k in range(1, NW):
            a = a + p_ref[k]
        sum_ref[...] = a[0:PSTR]
        m = a[PSTR:2 * PSTR]
        for w in range(1, POOLW):
            m = jnp.maximum(m, a[PSTR * (w + 1):PSTR * (w + 2)])
        max_ref[...] = m

    return pl.pallas_call(
        body,
        out_shape=[jax.ShapeDtypeStruct((PSTR,), jnp.float32),
                   jax.ShapeDtypeStruct((PSTR,), jnp.float32)],
    )(partials)


@jax.jit
def kernel(index, rate, starttime, endtime, t):
    tvec = jnp.full((16,), t, jnp.float32)
    parts = _sc_partials(index.astype(jnp.int32), rate, starttime,
                         endtime, tvec)
    s, m = _tc_combine(parts)
    return jnp.concatenate([s[:NBUCK], m[:NBUCK]])
